# baseline (speedup 1.0000x reference)
import functools

import jax
import jax.numpy as jnp
from jax import lax
from jax.experimental import pallas as pl
from jax.experimental.pallas import tpu as pltpu

_N, _J, _K = 4096, 1024, 16
_ROWS = _N * _J // 8
_RB = 8192


def _onehot_kernel(x_ref, o_ref):
    xv = x_ref[...]
    xc = jnp.clip(xv, 0.0, 15.5)
    idx = jnp.floor(xc).astype(jnp.bfloat16)

    bi = lax.broadcasted_iota(jnp.int32, (8, 128), 1)
    ci = lax.broadcasted_iota(jnp.int32, (8, 128), 0)
    sel = (bi // 16 == ci).astype(jnp.bfloat16)

    rep = lax.dot_general(idx, sel, (((1,), (0,)), ((), ())),
                          preferred_element_type=jnp.float32)

    kv = (lax.broadcasted_iota(jnp.int32, (1, 128), 1) % 16).astype(jnp.float32)
    o_ref[...] = (rep == kv).astype(jnp.float32)


@functools.partial(jax.jit, static_argnames=("interpret",))
def kernel(x, interpret=False):
    xr = x.reshape(_ROWS, 8)
    grid = (_ROWS // _RB,)
    out = pl.pallas_call(
        _onehot_kernel,
        grid=grid,
        in_specs=[pl.BlockSpec((_RB, 8), lambda g: (g, 0))],
        out_specs=pl.BlockSpec((_RB, 128), lambda g: (g, 0)),
        out_shape=jax.ShapeDtypeStruct((_ROWS, 128), jnp.float32),
        compiler_params=pltpu.CompilerParams(
            dimension_semantics=("arbitrary",),
        ),
        interpret=interpret,
    )(xr)
    return out.reshape(_N, _J, _K)


# speedup vs baseline: 3.6126x; 3.6126x over previous
import functools

import jax
import jax.numpy as jnp
from jax import lax
from jax.experimental import pallas as pl
from jax.experimental.pallas import tpu as pltpu

_N, _J, _K = 4096, 1024, 16
_R = 128


def _onehot_kernel(x_ref, o_ref):
    xv = x_ref[...]
    idx = jnp.floor(jnp.clip(xv, 0.0, 15.5)).astype(jnp.bfloat16)

    mi = lax.broadcasted_iota(jnp.int32, (128, 2048), 0)
    qi = lax.broadcasted_iota(jnp.int32, (128, 2048), 1)
    w = (mi == qi // 16).astype(jnp.bfloat16)
    kv = (lax.broadcasted_iota(jnp.int32, (1, 2048), 1) % 16).astype(
        jnp.float32)

    for c in range(8):
        part = idx[:, 128 * c:128 * (c + 1)]
        rep = lax.dot_general(part, w, (((1,), (0,)), ((), ())),
                              preferred_element_type=jnp.float32)
        o_ref[:, 2048 * c:2048 * (c + 1)] = (rep == kv).astype(jnp.float32)


@functools.partial(jax.jit, static_argnames=("interpret",))
def kernel(x, interpret=False):
    grid = (_N // _R,)
    out = pl.pallas_call(
        _onehot_kernel,
        grid=grid,
        in_specs=[pl.BlockSpec((_R, _J), lambda g: (g, 0))],
        out_specs=pl.BlockSpec((_R, _J * _K), lambda g: (g, 0)),
        out_shape=jax.ShapeDtypeStruct((_N, _J * _K), jnp.float32),
        compiler_params=pltpu.CompilerParams(
            dimension_semantics=("arbitrary",),
        ),
        interpret=interpret,
    )(x)
    return out.reshape(_N, _J, _K)


# speedup vs baseline: 19.2764x; 5.3359x over previous
import functools

import jax
import jax.numpy as jnp
from jax import lax
from jax.experimental import pallas as pl
from jax.experimental.pallas import tpu as pltpu

_N, _J, _K = 4096, 1024, 16
_R = 64


def _onehot_kernel(x_ref, o_ref):
    xv = x_ref[...]
    idx = jnp.floor(jnp.clip(xv, 0.0, 15.5)).astype(jnp.int32)
    ks = lax.broadcasted_iota(jnp.int32, (_R, _K, _J), 1)
    o_ref[...] = (idx[:, None, :] == ks).astype(jnp.float32)


@functools.partial(jax.jit, static_argnames=("interpret",))
def kernel(x, interpret=False):
    grid = (_N // _R,)
    out = pl.pallas_call(
        _onehot_kernel,
        grid=grid,
        in_specs=[pl.BlockSpec((_R, _J), lambda g: (g, 0))],
        out_specs=pl.BlockSpec((_R, _K, _J), lambda g: (g, 0, 0)),
        out_shape=jax.ShapeDtypeStruct((_N, _K, _J), jnp.float32),
        compiler_params=pltpu.CompilerParams(
            dimension_semantics=("arbitrary",),
        ),
        interpret=interpret,
    )(x)
    return jnp.transpose(out, (0, 2, 1))


# speedup vs baseline: 21.3185x; 1.1059x over previous
import functools

import jax
import jax.numpy as jnp
from jax import lax
from jax.experimental import pallas as pl
from jax.experimental.pallas import tpu as pltpu

_N, _J, _K = 4096, 1024, 16
_R = 128


def _onehot_kernel(x_ref, o_ref):
    xv = x_ref[...]
    idx = jnp.floor(jnp.clip(xv, 0.0, 15.5)).astype(jnp.int32)
    ks = lax.broadcasted_iota(jnp.int32, (_R, _K, _J), 1)
    o_ref[...] = (idx[:, None, :] == ks).astype(jnp.float32)


@functools.partial(jax.jit, static_argnames=("interpret",))
def kernel(x, interpret=False):
    grid = (_N // _R,)
    out = pl.pallas_call(
        _onehot_kernel,
        grid=grid,
        in_specs=[pl.BlockSpec((_R, _J), lambda g: (g, 0))],
        out_specs=pl.BlockSpec((_R, _K, _J), lambda g: (g, 0, 0)),
        out_shape=jax.ShapeDtypeStruct((_N, _K, _J), jnp.float32),
        compiler_params=pltpu.CompilerParams(
            dimension_semantics=("arbitrary",),
        ),
        interpret=interpret,
    )(x)
    return jnp.transpose(out, (0, 2, 1))


# speedup vs baseline: 21.3734x; 1.0026x over previous
import functools

import jax
import jax.numpy as jnp
from jax import lax
from jax.experimental import pallas as pl
from jax.experimental.pallas import tpu as pltpu

_N, _J, _K = 4096, 1024, 16
_R = 256


def _onehot_kernel(x_ref, o_ref):
    xv = x_ref[...]
    idx = jnp.floor(jnp.clip(xv, 0.0, 15.5)).astype(jnp.int32)
    ks = lax.broadcasted_iota(jnp.int32, (_R, _K, _J), 1)
    o_ref[...] = (idx[:, None, :] == ks).astype(jnp.float32)


@functools.partial(jax.jit, static_argnames=("interpret",))
def kernel(x, interpret=False):
    grid = (_N // _R,)
    out = pl.pallas_call(
        _onehot_kernel,
        grid=grid,
        in_specs=[pl.BlockSpec((_R, _J), lambda g: (g, 0))],
        out_specs=pl.BlockSpec((_R, _K, _J), lambda g: (g, 0, 0)),
        out_shape=jax.ShapeDtypeStruct((_N, _K, _J), jnp.float32),
        compiler_params=pltpu.CompilerParams(
            dimension_semantics=("arbitrary",),
        ),
        interpret=interpret,
    )(x)
    return jnp.transpose(out, (0, 2, 1))
